# baseline (speedup 1.0000x reference)
import jax
import jax.numpy as jnp
from jax.experimental import pallas as pl
from jax.experimental.pallas import tpu as pltpu

OBS_R = 0.25
HID_K = 64
BLK = 256

_HIGHEST = jax.lax.Precision.HIGHEST


def _mask_deg_kernel(rows_ref, posT_ref, mask_ref, deg_ref):
    rx = rows_ref[:, 0:1]
    ry = rows_ref[:, 1:2]
    ax = posT_ref[0:1, :]
    ay = posT_ref[1:2, :]
    dx = rx - ax
    dy = ry - ay
    dist = jnp.sqrt(dx * dx + dy * dy)
    m = dist <= OBS_R
    mask_ref[...] = m
    deg_ref[...] = jnp.sum(m.astype(jnp.float32), axis=1, keepdims=True)


def _split_z(z):
    hi = z.astype(jnp.bfloat16)
    lo = (z - hi.astype(jnp.float32)).astype(jnp.bfloat16)
    return hi, lo


def _convs_kernel(mask_ref, deg_ref, x_ref, w1_ref, b1_ref, w2_ref, b2_ref,
                  wc_ref, bc_ref, out_ref, h1_ref, zhi_ref, zlo_ref):
    p = pl.program_id(0)
    i = pl.program_id(1)

    @pl.when((p == 0) & (i == 0))
    def _():
        dis = 1.0 / jnp.sqrt(deg_ref[...])
        xw = jnp.dot(x_ref[...], w1_ref[...], preferred_element_type=jnp.float32,
                     precision=_HIGHEST)
        zhi_ref[...], zlo_ref[...] = _split_z(xw * dis)

    @pl.when((p == 1) & (i == 0))
    def _():
        dis = 1.0 / jnp.sqrt(deg_ref[...])
        hw = jnp.dot(h1_ref[...], w2_ref[...], preferred_element_type=jnp.float32,
                     precision=_HIGHEST)
        zhi_ref[...], zlo_ref[...] = _split_z(hw * dis)

    mbf = mask_ref[...].astype(jnp.bfloat16)
    agg = (jnp.dot(mbf, zhi_ref[...], preferred_element_type=jnp.float32)
           + jnp.dot(mbf, zlo_ref[...], preferred_element_type=jnp.float32))
    dis_blk = 1.0 / jnp.sqrt(deg_ref[pl.ds(i * BLK, BLK), :])

    @pl.when(p == 0)
    def _():
        h1_ref[pl.ds(i * BLK, BLK), :] = jnp.maximum(
            agg * dis_blk + b1_ref[...], 0.0)
        out_ref[...] = jnp.zeros_like(out_ref)

    @pl.when(p == 1)
    def _():
        h2 = jnp.maximum(agg * dis_blk + b2_ref[...], 0.0)
        out_ref[...] = jnp.dot(h2, wc_ref[...],
                               preferred_element_type=jnp.float32,
                               precision=_HIGHEST) + bc_ref[...]


def kernel(states, W1, b1, W2, b2, Wc, bc):
    n = states.shape[0]
    in_dim = states.shape[1]
    nblk = n // BLK
    posT = jnp.zeros((8, n), jnp.float32)
    posT = posT.at[0].set(states[:, 0]).at[1].set(states[:, 1])

    mask, deg = pl.pallas_call(
        _mask_deg_kernel,
        grid=(nblk,),
        in_specs=[
            pl.BlockSpec((BLK, in_dim), lambda i: (i, 0)),
            pl.BlockSpec((8, n), lambda i: (0, 0)),
        ],
        out_specs=[
            pl.BlockSpec((BLK, n), lambda i: (i, 0)),
            pl.BlockSpec((BLK, 1), lambda i: (i, 0)),
        ],
        out_shape=[
            jax.ShapeDtypeStruct((n, n), jnp.bool_),
            jax.ShapeDtypeStruct((n, 1), jnp.float32),
        ],
    )(states, posT)

    h = pl.pallas_call(
        _convs_kernel,
        grid=(2, nblk),
        in_specs=[
            pl.BlockSpec((BLK, n), lambda p, i: (i, 0)),
            pl.BlockSpec((n, 1), lambda p, i: (0, 0)),
            pl.BlockSpec((n, in_dim), lambda p, i: (0, 0)),
            pl.BlockSpec((in_dim, HID_K), lambda p, i: (0, 0)),
            pl.BlockSpec((1, HID_K), lambda p, i: (0, 0)),
            pl.BlockSpec((HID_K, HID_K), lambda p, i: (0, 0)),
            pl.BlockSpec((1, HID_K), lambda p, i: (0, 0)),
            pl.BlockSpec((HID_K, 1), lambda p, i: (0, 0)),
            pl.BlockSpec((1, 1), lambda p, i: (0, 0)),
        ],
        out_specs=pl.BlockSpec((BLK, 1), lambda p, i: (i, 0)),
        out_shape=jax.ShapeDtypeStruct((n, 1), jnp.float32),
        scratch_shapes=[pltpu.VMEM((n, HID_K), jnp.float32),
                        pltpu.VMEM((n, HID_K), jnp.bfloat16),
                        pltpu.VMEM((n, HID_K), jnp.bfloat16)],
    )(mask, deg, states, W1, b1.reshape(1, HID_K), W2,
      b2.reshape(1, HID_K), Wc, bc.reshape(1, 1))

    return (h, mask)


# speedup vs baseline: 1.1382x; 1.1382x over previous
import jax
import jax.numpy as jnp
from jax.experimental import pallas as pl
from jax.experimental.pallas import tpu as pltpu

OBS_R = 0.25
HID_K = 64
BLK = 256

_HIGHEST = jax.lax.Precision.HIGHEST


def _mask_deg_kernel(rows_ref, posT_ref, mask_ref, maskbf_ref, deg_ref):
    rx = rows_ref[:, 0:1]
    ry = rows_ref[:, 1:2]
    ax = posT_ref[0:1, :]
    ay = posT_ref[1:2, :]
    dx = rx - ax
    dy = ry - ay
    dist = jnp.sqrt(dx * dx + dy * dy)
    m = dist <= OBS_R
    mask_ref[...] = m
    maskbf_ref[...] = m.astype(jnp.bfloat16)
    deg_ref[...] = jnp.sum(m.astype(jnp.float32), axis=1, keepdims=True)


def _convs_kernel(maskbf_ref, deg_ref, x_ref, w1_ref, b1_ref, w2_ref, b2_ref,
                  wc_ref, bc_ref, out_ref, h1_ref, zhi_ref):
    p = pl.program_id(0)
    i = pl.program_id(1)

    @pl.when((p == 0) & (i == 0))
    def _():
        dis = 1.0 / jnp.sqrt(deg_ref[...])
        xw = jnp.dot(x_ref[...], w1_ref[...], preferred_element_type=jnp.float32,
                     precision=_HIGHEST)
        zhi_ref[...] = (xw * dis).astype(jnp.bfloat16)

    @pl.when((p == 1) & (i == 0))
    def _():
        dis = 1.0 / jnp.sqrt(deg_ref[...])
        hw = jnp.dot(h1_ref[...], w2_ref[...], preferred_element_type=jnp.float32,
                     precision=_HIGHEST)
        zhi_ref[...] = (hw * dis).astype(jnp.bfloat16)

    mbf = maskbf_ref[...]
    agg = jnp.dot(mbf, zhi_ref[...], preferred_element_type=jnp.float32)
    dis_blk = 1.0 / jnp.sqrt(deg_ref[pl.ds(i * BLK, BLK), :])

    @pl.when(p == 0)
    def _():
        h1_ref[pl.ds(i * BLK, BLK), :] = jnp.maximum(
            agg * dis_blk + b1_ref[...], 0.0)
        out_ref[...] = jnp.zeros_like(out_ref)

    @pl.when(p == 1)
    def _():
        h2 = jnp.maximum(agg * dis_blk + b2_ref[...], 0.0)
        out_ref[...] = jnp.dot(h2, wc_ref[...],
                               preferred_element_type=jnp.float32,
                               precision=_HIGHEST) + bc_ref[...]


def kernel(states, W1, b1, W2, b2, Wc, bc):
    n = states.shape[0]
    in_dim = states.shape[1]
    nblk = n // BLK
    posT = jnp.zeros((8, n), jnp.float32)
    posT = posT.at[0].set(states[:, 0]).at[1].set(states[:, 1])

    mask, maskbf, deg = pl.pallas_call(
        _mask_deg_kernel,
        grid=(nblk,),
        in_specs=[
            pl.BlockSpec((BLK, in_dim), lambda i: (i, 0)),
            pl.BlockSpec((8, n), lambda i: (0, 0)),
        ],
        out_specs=[
            pl.BlockSpec((BLK, n), lambda i: (i, 0)),
            pl.BlockSpec((BLK, n), lambda i: (i, 0)),
            pl.BlockSpec((BLK, 1), lambda i: (i, 0)),
        ],
        out_shape=[
            jax.ShapeDtypeStruct((n, n), jnp.bool_),
            jax.ShapeDtypeStruct((n, n), jnp.bfloat16),
            jax.ShapeDtypeStruct((n, 1), jnp.float32),
        ],
    )(states, posT)

    h = pl.pallas_call(
        _convs_kernel,
        grid=(2, nblk),
        in_specs=[
            pl.BlockSpec((BLK, n), lambda p, i: (i, 0)),
            pl.BlockSpec((n, 1), lambda p, i: (0, 0)),
            pl.BlockSpec((n, in_dim), lambda p, i: (0, 0)),
            pl.BlockSpec((in_dim, HID_K), lambda p, i: (0, 0)),
            pl.BlockSpec((1, HID_K), lambda p, i: (0, 0)),
            pl.BlockSpec((HID_K, HID_K), lambda p, i: (0, 0)),
            pl.BlockSpec((1, HID_K), lambda p, i: (0, 0)),
            pl.BlockSpec((HID_K, 1), lambda p, i: (0, 0)),
            pl.BlockSpec((1, 1), lambda p, i: (0, 0)),
        ],
        out_specs=pl.BlockSpec((BLK, 1), lambda p, i: (i, 0)),
        out_shape=jax.ShapeDtypeStruct((n, 1), jnp.float32),
        scratch_shapes=[pltpu.VMEM((n, HID_K), jnp.float32),
                        pltpu.VMEM((n, HID_K), jnp.bfloat16)],
    )(maskbf, deg, states, W1, b1.reshape(1, HID_K), W2,
      b2.reshape(1, HID_K), Wc, bc.reshape(1, 1))

    return (h, mask)


# speedup vs baseline: 1.2604x; 1.1073x over previous
import jax
import jax.numpy as jnp
from jax.experimental import pallas as pl
from jax.experimental.pallas import tpu as pltpu

OBS_R = 0.25
HID_K = 64
BLK = 512

_HIGHEST = jax.lax.Precision.HIGHEST


def _mask_deg_kernel(rows_ref, posT_ref, mask_ref, maskbf_ref, deg_ref):
    rx = rows_ref[:, 0:1]
    ry = rows_ref[:, 1:2]
    ax = posT_ref[0:1, :]
    ay = posT_ref[1:2, :]
    dx = rx - ax
    dy = ry - ay
    dist = jnp.sqrt(dx * dx + dy * dy)
    m = dist <= OBS_R
    mask_ref[...] = m
    maskbf_ref[...] = m.astype(jnp.bfloat16)
    deg_ref[...] = jnp.sum(m.astype(jnp.float32), axis=1, keepdims=True)


def _convs_kernel(maskbf_ref, deg_ref, x_ref, w1_ref, b1_ref, w2_ref, b2_ref,
                  wc_ref, bc_ref, out_ref, h1_ref, zhi_ref, dis_ref):
    p = pl.program_id(0)
    i = pl.program_id(1)

    @pl.when((p == 0) & (i == 0))
    def _():
        dis = 1.0 / jnp.sqrt(deg_ref[...])
        dis_ref[...] = dis
        xw = jnp.dot(x_ref[...], w1_ref[...], preferred_element_type=jnp.float32,
                     precision=_HIGHEST)
        zhi_ref[...] = (xw * dis).astype(jnp.bfloat16)

    @pl.when((p == 1) & (i == 0))
    def _():
        hw = jnp.dot(h1_ref[...], w2_ref[...], preferred_element_type=jnp.float32,
                     precision=_HIGHEST)
        zhi_ref[...] = (hw * dis_ref[...]).astype(jnp.bfloat16)

    mbf = maskbf_ref[...]
    agg = jnp.dot(mbf, zhi_ref[...], preferred_element_type=jnp.float32)
    dis_blk = dis_ref[pl.ds(i * BLK, BLK), :]

    @pl.when(p == 0)
    def _():
        h1_ref[pl.ds(i * BLK, BLK), :] = jnp.maximum(
            agg * dis_blk + b1_ref[...], 0.0)
        out_ref[...] = jnp.zeros_like(out_ref)

    @pl.when(p == 1)
    def _():
        h2 = jnp.maximum(agg * dis_blk + b2_ref[...], 0.0)
        out_ref[...] = jnp.dot(h2, wc_ref[...],
                               preferred_element_type=jnp.float32,
                               precision=_HIGHEST) + bc_ref[...]


def kernel(states, W1, b1, W2, b2, Wc, bc):
    n = states.shape[0]
    in_dim = states.shape[1]
    nblk = n // BLK
    posT = jnp.zeros((8, n), jnp.float32)
    posT = posT.at[0].set(states[:, 0]).at[1].set(states[:, 1])

    mask, maskbf, deg = pl.pallas_call(
        _mask_deg_kernel,
        grid=(nblk,),
        in_specs=[
            pl.BlockSpec((BLK, in_dim), lambda i: (i, 0)),
            pl.BlockSpec((8, n), lambda i: (0, 0)),
        ],
        out_specs=[
            pl.BlockSpec((BLK, n), lambda i: (i, 0)),
            pl.BlockSpec((BLK, n), lambda i: (i, 0)),
            pl.BlockSpec((BLK, 1), lambda i: (i, 0)),
        ],
        out_shape=[
            jax.ShapeDtypeStruct((n, n), jnp.bool_),
            jax.ShapeDtypeStruct((n, n), jnp.bfloat16),
            jax.ShapeDtypeStruct((n, 1), jnp.float32),
        ],
    )(states, posT)

    h = pl.pallas_call(
        _convs_kernel,
        grid=(2, nblk),
        in_specs=[
            pl.BlockSpec((BLK, n), lambda p, i: (i, 0)),
            pl.BlockSpec((n, 1), lambda p, i: (0, 0)),
            pl.BlockSpec((n, in_dim), lambda p, i: (0, 0)),
            pl.BlockSpec((in_dim, HID_K), lambda p, i: (0, 0)),
            pl.BlockSpec((1, HID_K), lambda p, i: (0, 0)),
            pl.BlockSpec((HID_K, HID_K), lambda p, i: (0, 0)),
            pl.BlockSpec((1, HID_K), lambda p, i: (0, 0)),
            pl.BlockSpec((HID_K, 1), lambda p, i: (0, 0)),
            pl.BlockSpec((1, 1), lambda p, i: (0, 0)),
        ],
        out_specs=pl.BlockSpec((BLK, 1), lambda p, i: (i, 0)),
        out_shape=jax.ShapeDtypeStruct((n, 1), jnp.float32),
        scratch_shapes=[pltpu.VMEM((n, HID_K), jnp.float32),
                        pltpu.VMEM((n, HID_K), jnp.bfloat16),
                        pltpu.VMEM((n, 1), jnp.float32)],
    )(maskbf, deg, states, W1, b1.reshape(1, HID_K), W2,
      b2.reshape(1, HID_K), Wc, bc.reshape(1, 1))

    return (h, mask)
